# baseline (speedup 1.0000x reference)
import functools

import jax
import jax.numpy as jnp
from jax import lax
from jax.experimental import pallas as pl
from jax.experimental.pallas import tpu as pltpu
from jax.experimental.pallas import tpu_sc as plsc

N = 10000
IN_C = 128
OUT_C = 256
K = 16
M = 2500

NPAD = 10240
MPAD = 2528
MSC = 2560

_BIGF = float(1e30)
_NEGF = float(-1e30)
_BIGI = 2**30


def _bfly_max(v):
    for sh in (1, 2, 4, 8, 16, 32, 64):
        v = jnp.maximum(v, jnp.roll(v, sh, axis=1))
    for sh in (1, 2, 4):
        v = jnp.maximum(v, jnp.roll(v, sh, axis=0))
    return v


def _bfly_min(v):
    for sh in (1, 2, 4, 8, 16, 32, 64):
        v = jnp.minimum(v, jnp.roll(v, sh, axis=1))
    for sh in (1, 2, 4):
        v = jnp.minimum(v, jnp.roll(v, sh, axis=0))
    return v


def _fps_body(px_ref, py_ref, pz_ref, out_ref, dists_ref):
    i = pl.program_id(0)
    vv = jax.lax.broadcasted_iota(jnp.int32, (10, 8, 128), 0)
    ss = jax.lax.broadcasted_iota(jnp.int32, (10, 8, 128), 1)
    ll = jax.lax.broadcasted_iota(jnp.int32, (10, 8, 128), 2)
    flat = ss * 1280 + vv * 128 + ll
    valid = flat < N

    @pl.when(i == 0)
    def _init():
        dists_ref[:, :, :] = jnp.where(valid, _BIGF, _NEGF)

    iota8 = jax.lax.broadcasted_iota(jnp.int32, (8, 1), 0)

    def step(t, carry):
        sx_a, sy_a, sz_a = carry
        dists = dists_ref[:, :, :]
        m = _bfly_max(jnp.max(dists, axis=0))
        cand = jnp.where(dists == m[None], flat, _BIGI)
        nxt = _bfly_min(jnp.min(cand, axis=0))
        selmask = flat == nxt[None]
        px = px_ref[:, :, :]
        py = py_ref[:, :, :]
        pz = pz_ref[:, :, :]
        sx = _bfly_max(jnp.max(jnp.where(selmask, px, -1.0), axis=0))
        sy = _bfly_max(jnp.max(jnp.where(selmask, py, -1.0), axis=0))
        sz = _bfly_max(jnp.max(jnp.where(selmask, pz, -1.0), axis=0))
        dx = px - sx[None]
        dy = py - sy[None]
        dz = pz - sz[None]
        d = dx * dx + dy * dy + dz * dz
        dists_ref[:, :, :] = jnp.minimum(dists, d)
        sx_a = jnp.where(iota8 == t, sx[:, 0:1], sx_a)
        sy_a = jnp.where(iota8 == t, sy[:, 0:1], sy_a)
        sz_a = jnp.where(iota8 == t, sz[:, 0:1], sz_a)
        return (sx_a, sy_a, sz_a)

    z = jnp.zeros((8, 1), jnp.float32)
    sx_a, sy_a, sz_a = lax.fori_loop(0, 8, step, (z, z, z))
    out_ref[:, :] = jnp.concatenate([sx_a, sy_a, sz_a], axis=1)


def _run_fps(px, py, pz):
    return pl.pallas_call(
        _fps_body,
        grid=(MPAD // 8,),
        in_specs=[
            pl.BlockSpec((10, 8, 128), lambda i: (0, 0, 0)),
            pl.BlockSpec((10, 8, 128), lambda i: (0, 0, 0)),
            pl.BlockSpec((10, 8, 128), lambda i: (0, 0, 0)),
        ],
        out_specs=pl.BlockSpec((8, 3), lambda i: (i, 0)),
        out_shape=jax.ShapeDtypeStruct((MPAD, 3), jnp.float32),
        scratch_shapes=[pltpu.VMEM((10, 8, 128), jnp.float32)],
    )(px, py, pz)


def _mlp_body(x_ref, w_ref, b_ref, out_ref):
    acc = jnp.dot(x_ref[:, :], w_ref[:, :], preferred_element_type=jnp.float32)
    out_ref[:, :] = jnp.maximum(acc + b_ref[:, :], 0.0)


def _run_mlp(x, w, b2):
    return pl.pallas_call(
        _mlp_body,
        grid=(10,),
        in_specs=[
            pl.BlockSpec((1000, IN_C), lambda i: (i, 0)),
            pl.BlockSpec((IN_C, OUT_C), lambda i: (0, 0)),
            pl.BlockSpec((1, OUT_C), lambda i: (0, 0)),
        ],
        out_specs=pl.BlockSpec((1000, OUT_C), lambda i: (i, 0)),
        out_shape=jax.ShapeDtypeStruct((N, OUT_C), jnp.float32),
    )(x, w, b2)


def _knn_body(pt_ref, q_ref, out_ref):
    qblk = q_ref[:, :]
    qx = qblk[:, 0:1]
    qy = qblk[:, 1:2]
    qz = qblk[:, 2:3]
    px = pt_ref[0:1, :]
    py = pt_ref[1:2, :]
    pz = pt_ref[2:3, :]
    ynorm = qx * qx + qy * qy + qz * qz
    xnorm = px * px + py * py + pz * pz
    g = jnp.dot(qblk, pt_ref[:, :], preferred_element_type=jnp.float32)
    neg = -((ynorm - 2.0 * g) + xnorm)
    colmap = jax.lax.broadcasted_iota(jnp.int32, (QB, NPAD), 1)
    cols = []
    for _ in range(K):
        m = jnp.max(neg, axis=1, keepdims=True)
        c = jnp.min(jnp.where(neg == m, colmap, _BIGI), axis=1, keepdims=True)
        cols.append(c)
        neg = jnp.where(colmap == c, _NEGF, neg)
    out_ref[:, :] = jnp.concatenate(cols, axis=1)


QB = 32


def _run_knn(pt, q):
    return pl.pallas_call(
        _knn_body,
        grid=(MPAD // QB,),
        in_specs=[
            pl.BlockSpec((3, NPAD), lambda i: (0, 0)),
            pl.BlockSpec((QB, 3), lambda i: (i, 0)),
        ],
        out_specs=pl.BlockSpec((QB, K), lambda i: (i, 0)),
        out_shape=jax.ShapeDtypeStruct((MPAD, K), jnp.int32),
    )(pt, q)


_Q_PER_TILE = MSC // 32
_Q_CHUNK = 8
_N_CHUNKS = _Q_PER_TILE // _Q_CHUNK


def _sc_gather_max(h, idx_flat):
    info = plsc.get_sparse_core_info()
    nc = info.num_cores

    mesh = plsc.VectorSubcoreMesh(core_axis_name="c", subcore_axis_name="s")

    @functools.partial(
        pl.kernel,
        mesh=mesh,
        out_type=jax.ShapeDtypeStruct((MSC, OUT_C), jnp.float32),
        scratch_types=[
            pltpu.VMEM((_Q_CHUNK * K,), jnp.int32),
            pltpu.VMEM((_Q_CHUNK * K, OUT_C), jnp.float32),
            pltpu.VMEM((_Q_CHUNK, OUT_C), jnp.float32),
            pltpu.SemaphoreType.DMA,
        ],
    )
    def k(h_hbm, idx_hbm, out_hbm, idx_v, rows_v, acc_v, sem):
        wid = lax.axis_index("s") * nc + lax.axis_index("c")
        base_q = wid * _Q_PER_TILE

        def chunk(c, _):
            qoff = base_q + c * _Q_CHUNK
            pltpu.sync_copy(idx_hbm.at[pl.ds(qoff * K, _Q_CHUNK * K)], idx_v)
            pltpu.async_copy(h_hbm.at[idx_v], rows_v, sem).wait()

            def per_query(q, __):
                row0 = q * K
                for d in range(OUT_C // 16):
                    ds = pl.ds(d * 16, 16)
                    acc = rows_v[row0, ds]
                    for r in range(1, K):
                        acc = jnp.maximum(acc, rows_v[row0 + r, ds])
                    acc_v[q, ds] = acc
                return 0

            lax.fori_loop(0, _Q_CHUNK, per_query, 0)
            pltpu.sync_copy(acc_v, out_hbm.at[pl.ds(qoff, _Q_CHUNK)])
            return 0

        lax.fori_loop(0, _N_CHUNKS, chunk, 0)

    return k(h, idx_flat)


def kernel(x, pos, batch, W, b):
    posx = jnp.pad(pos[:, 0], (0, NPAD - N), constant_values=1e3)
    posy = jnp.pad(pos[:, 1], (0, NPAD - N), constant_values=1e3)
    posz = jnp.pad(pos[:, 2], (0, NPAD - N), constant_values=1e3)
    px = posx.reshape(8, 10, 128).transpose(1, 0, 2)
    py = posy.reshape(8, 10, 128).transpose(1, 0, 2)
    pz = posz.reshape(8, 10, 128).transpose(1, 0, 2)
    pt = jnp.stack([posx, posy, posz])

    subpos_pad = _run_fps(px, py, pz)
    sub_pos = subpos_pad[:M]

    h = _run_mlp(x, W, b.reshape(1, OUT_C))

    idx_pad = _run_knn(pt, subpos_pad)
    idx = idx_pad[:M]

    idx_sc = jnp.zeros((MSC, K), jnp.int32).at[:M].set(idx).reshape(-1)
    x_out = _sc_gather_max(h, idx_sc)[:M]

    sub_batch = jnp.zeros((M,), jnp.int32)
    return (x_out, sub_pos, sub_batch)


# speedup vs baseline: 2.3876x; 2.3876x over previous
import functools

import jax
import jax.numpy as jnp
from jax import lax
from jax.experimental import pallas as pl
from jax.experimental.pallas import tpu as pltpu
from jax.experimental.pallas import tpu_sc as plsc

N = 10000
IN_C = 128
OUT_C = 256
K = 16
M = 2500

NPAD = 10240
MPAD = 2528
MSC = 2560

_BIGF = float(1e30)
_NEGF = float(-1e30)
_BIGI = 2**30


def _fused_body(px_ref, py_ref, pz_ref, pt_ref, out_ref, idx_ref, dists_ref, qsave_ref):
    j = pl.program_id(0)
    rows = jax.lax.broadcasted_iota(jnp.int32, (8, 1280), 0)
    cols = jax.lax.broadcasted_iota(jnp.int32, (8, 1280), 1)
    flat = rows * 1280 + cols
    valid = flat < N

    @pl.when(j == 0)
    def _init():
        dists_ref[:, :] = jnp.where(valid, _BIGF, _NEGF)

    qprev = qsave_ref[:, :]

    iotaq = jax.lax.broadcasted_iota(jnp.int32, (QB, 1), 0)

    @pl.when(j < MPAD // QB)
    def _fps():
        def step(t, carry):
            sx_a, sy_a, sz_a = carry
            dists = dists_ref[:, :]
            m = jnp.max(dists)
            nxt = jnp.min(jnp.where(dists == m, flat, _BIGI))
            selmask = flat == nxt
            px = px_ref[:, :]
            py = py_ref[:, :]
            pz = pz_ref[:, :]
            sx = jnp.sum(jnp.where(selmask, px, 0.0))
            sy = jnp.sum(jnp.where(selmask, py, 0.0))
            sz = jnp.sum(jnp.where(selmask, pz, 0.0))
            dx = px - sx
            dy = py - sy
            dz = pz - sz
            d = dx * dx + dy * dy + dz * dz
            dists_ref[:, :] = jnp.minimum(dists, d)
            sx_a = jnp.where(iotaq == t, sx, sx_a)
            sy_a = jnp.where(iotaq == t, sy, sy_a)
            sz_a = jnp.where(iotaq == t, sz, sz_a)
            return (sx_a, sy_a, sz_a)

        z = jnp.zeros((QB, 1), jnp.float32)
        sx_a, sy_a, sz_a = lax.fori_loop(0, QB, step, (z, z, z))
        qnew = jnp.concatenate([sx_a, sy_a, sz_a], axis=1)
        out_ref[:, :] = qnew
        qsave_ref[:, :] = qnew

    @pl.when(j >= 1)
    def _knn():
        qx = qprev[:, 0:1]
        qy = qprev[:, 1:2]
        qz = qprev[:, 2:3]
        ptx = pt_ref[0:1, :]
        pty = pt_ref[1:2, :]
        ptz = pt_ref[2:3, :]
        ynorm = qx * qx + qy * qy + qz * qz
        xnorm = ptx * ptx + pty * pty + ptz * ptz
        g = jnp.dot(qprev, pt_ref[:, :], preferred_element_type=jnp.float32)
        neg = -((ynorm - 2.0 * g) + xnorm)
        colmap = jax.lax.broadcasted_iota(jnp.int32, (QB, NPAD), 1)
        sel = []
        for _ in range(K):
            m = jnp.max(neg, axis=1, keepdims=True)
            c = jnp.min(jnp.where(neg == m, colmap, _BIGI), axis=1, keepdims=True)
            sel.append(c)
            neg = jnp.where(colmap == c, _NEGF, neg)
        idx_ref[:, :] = jnp.concatenate(sel, axis=1)


def _run_fps_knn(px, py, pz, pt):
    nblk = MPAD // QB
    return pl.pallas_call(
        _fused_body,
        grid=(nblk + 1,),
        in_specs=[
            pl.BlockSpec((8, 1280), lambda j: (0, 0)),
            pl.BlockSpec((8, 1280), lambda j: (0, 0)),
            pl.BlockSpec((8, 1280), lambda j: (0, 0)),
            pl.BlockSpec((3, NPAD), lambda j: (0, 0)),
        ],
        out_specs=[
            pl.BlockSpec((QB, 3), lambda j: (jnp.minimum(j, nblk - 1), 0)),
            pl.BlockSpec((QB, K), lambda j: (jnp.maximum(j - 1, 0), 0)),
        ],
        out_shape=[
            jax.ShapeDtypeStruct((MPAD, 3), jnp.float32),
            jax.ShapeDtypeStruct((MPAD, K), jnp.int32),
        ],
        scratch_shapes=[
            pltpu.VMEM((8, 1280), jnp.float32),
            pltpu.VMEM((QB, 3), jnp.float32),
        ],
    )(px, py, pz, pt)


QB = 32


def _mlp_body(x_ref, w_ref, b_ref, out_ref):
    acc = jnp.dot(x_ref[:, :], w_ref[:, :], preferred_element_type=jnp.float32)
    out_ref[:, :] = jnp.maximum(acc + b_ref[:, :], 0.0)


def _run_mlp(x, w, b2):
    return pl.pallas_call(
        _mlp_body,
        grid=(10,),
        in_specs=[
            pl.BlockSpec((1000, IN_C), lambda i: (i, 0)),
            pl.BlockSpec((IN_C, OUT_C), lambda i: (0, 0)),
            pl.BlockSpec((1, OUT_C), lambda i: (0, 0)),
        ],
        out_specs=pl.BlockSpec((1000, OUT_C), lambda i: (i, 0)),
        out_shape=jax.ShapeDtypeStruct((N, OUT_C), jnp.float32),
    )(x, w, b2)


_Q_PER_TILE = MSC // 32
_Q_CHUNK = 8
_N_CHUNKS = _Q_PER_TILE // _Q_CHUNK


def _sc_gather_max(h, idx_flat):
    info = plsc.get_sparse_core_info()
    nc = info.num_cores

    mesh = plsc.VectorSubcoreMesh(core_axis_name="c", subcore_axis_name="s")

    @functools.partial(
        pl.kernel,
        mesh=mesh,
        out_type=jax.ShapeDtypeStruct((MSC, OUT_C), jnp.float32),
        scratch_types=[
            pltpu.VMEM((_Q_CHUNK * K,), jnp.int32),
            pltpu.VMEM((_Q_CHUNK * K, OUT_C), jnp.float32),
            pltpu.VMEM((_Q_CHUNK, OUT_C), jnp.float32),
            pltpu.SemaphoreType.DMA,
        ],
    )
    def k(h_hbm, idx_hbm, out_hbm, idx_v, rows_v, acc_v, sem):
        wid = lax.axis_index("s") * nc + lax.axis_index("c")
        base_q = wid * _Q_PER_TILE

        def chunk(c, _):
            qoff = base_q + c * _Q_CHUNK
            pltpu.sync_copy(idx_hbm.at[pl.ds(qoff * K, _Q_CHUNK * K)], idx_v)
            pltpu.async_copy(h_hbm.at[idx_v], rows_v, sem).wait()

            def per_query(q, __):
                row0 = q * K
                for d in range(OUT_C // 16):
                    ds = pl.ds(d * 16, 16)
                    acc = rows_v[row0, ds]
                    for r in range(1, K):
                        acc = jnp.maximum(acc, rows_v[row0 + r, ds])
                    acc_v[q, ds] = acc
                return 0

            lax.fori_loop(0, _Q_CHUNK, per_query, 0)
            pltpu.sync_copy(acc_v, out_hbm.at[pl.ds(qoff, _Q_CHUNK)])
            return 0

        lax.fori_loop(0, _N_CHUNKS, chunk, 0)

    return k(h, idx_flat)


def kernel(x, pos, batch, W, b):
    posx = jnp.pad(pos[:, 0], (0, NPAD - N), constant_values=1e3)
    posy = jnp.pad(pos[:, 1], (0, NPAD - N), constant_values=1e3)
    posz = jnp.pad(pos[:, 2], (0, NPAD - N), constant_values=1e3)
    px = posx.reshape(8, 1280)
    py = posy.reshape(8, 1280)
    pz = posz.reshape(8, 1280)
    pt = jnp.stack([posx, posy, posz])

    h = _run_mlp(x, W, b.reshape(1, OUT_C))

    subpos_pad, idx_pad = _run_fps_knn(px, py, pz, pt)
    sub_pos = subpos_pad[:M]
    idx = idx_pad[:M]

    idx_sc = jnp.zeros((MSC, K), jnp.int32).at[:M].set(idx).reshape(-1)
    x_out = _sc_gather_max(h, idx_sc)[:M]

    sub_batch = jnp.zeros((M,), jnp.int32)
    return (x_out, sub_pos, sub_batch)


# speedup vs baseline: 2.9123x; 1.2198x over previous
import functools

import jax
import jax.numpy as jnp
from jax import lax
from jax.experimental import pallas as pl
from jax.experimental.pallas import tpu as pltpu
from jax.experimental.pallas import tpu_sc as plsc

N = 10000
IN_C = 128
OUT_C = 256
K = 16
M = 2500

NPAD = 10240
MPAD = 2528
MSC = 2560

_BIGF = float(1e30)
_NEGF = float(-1e30)
_BIGI = 2**30


def _fps_step(px_ref, py_ref, pz_ref, dists_ref, flat, iotaq, t, carry):
    sx_a, sy_a, sz_a = carry
    dists = dists_ref[:, :]
    m = jnp.max(dists)
    nxt = jnp.min(jnp.where(dists == m, flat, _BIGI))
    selmask = flat == nxt
    px = px_ref[:, :]
    py = py_ref[:, :]
    pz = pz_ref[:, :]
    sx = jnp.sum(jnp.where(selmask, px, 0.0))
    sy = jnp.sum(jnp.where(selmask, py, 0.0))
    sz = jnp.sum(jnp.where(selmask, pz, 0.0))
    dx = px - sx
    dy = py - sy
    dz = pz - sz
    d = dx * dx + dy * dy + dz * dz
    dists_ref[:, :] = jnp.minimum(dists, d)
    sx_a = jnp.where(iotaq == t, sx, sx_a)
    sy_a = jnp.where(iotaq == t, sy, sy_a)
    sz_a = jnp.where(iotaq == t, sz, sz_a)
    return (sx_a, sy_a, sz_a)


def _fused_body(px_ref, py_ref, pz_ref, pt_ref, out_ref, idx_ref,
                dists_ref, qsave_ref, neg_ref):
    j = pl.program_id(0)
    nblk = MPAD // QB
    rows = jax.lax.broadcasted_iota(jnp.int32, (8, 1280), 0)
    cols = jax.lax.broadcasted_iota(jnp.int32, (8, 1280), 1)
    flat = rows * 1280 + cols
    valid = flat < N

    @pl.when(j == 0)
    def _init():
        dists_ref[:, :] = jnp.where(valid, _BIGF, _NEGF)

    qprev = qsave_ref[:, :]

    qx = qprev[:, 0:1]
    qy = qprev[:, 1:2]
    qz = qprev[:, 2:3]
    ptx = pt_ref[0:1, :]
    pty = pt_ref[1:2, :]
    ptz = pt_ref[2:3, :]
    ynorm = qx * qx + qy * qy + qz * qz
    xnorm = ptx * ptx + pty * pty + ptz * ptz
    g = jnp.dot(qprev, pt_ref[:, :], preferred_element_type=jnp.float32)
    neg_ref[:, :] = -((ynorm - 2.0 * g) + xnorm)

    iotaq = jax.lax.broadcasted_iota(jnp.int32, (QB, 1), 0)
    colmap = jax.lax.broadcasted_iota(jnp.int32, (QB, NPAD), 1)
    lanek = jax.lax.broadcasted_iota(jnp.int32, (QB, K), 1)

    def step(t, carry):
        sx_a, sy_a, sz_a, idxacc = carry
        fps_c = (sx_a, sy_a, sz_a)
        fps_c = _fps_step(px_ref, py_ref, pz_ref, dists_ref, flat, iotaq,
                          2 * t, fps_c)
        fps_c = _fps_step(px_ref, py_ref, pz_ref, dists_ref, flat, iotaq,
                          2 * t + 1, fps_c)
        sx_a, sy_a, sz_a = fps_c
        neg = neg_ref[:, :]
        m = jnp.max(neg, axis=1, keepdims=True)
        c = jnp.min(jnp.where(neg == m, colmap, _BIGI), axis=1, keepdims=True)
        neg_ref[:, :] = jnp.where(colmap == c, _NEGF, neg)
        idxacc = jnp.where(lanek == t, c, idxacc)
        return (sx_a, sy_a, sz_a, idxacc)

    z = jnp.zeros((QB, 1), jnp.float32)
    zi = jnp.zeros((QB, K), jnp.int32)
    sx_a, sy_a, sz_a, idxacc = lax.fori_loop(0, K, step, (z, z, z, zi))

    @pl.when(j < nblk)
    def _wq():
        qnew = jnp.concatenate([sx_a, sy_a, sz_a], axis=1)
        out_ref[:, :] = qnew
        qsave_ref[:, :] = qnew

    @pl.when(j >= 1)
    def _wi():
        idx_ref[:, :] = idxacc


def _run_fps_knn(px, py, pz, pt):
    nblk = MPAD // QB
    return pl.pallas_call(
        _fused_body,
        grid=(nblk + 1,),
        in_specs=[
            pl.BlockSpec((8, 1280), lambda j: (0, 0)),
            pl.BlockSpec((8, 1280), lambda j: (0, 0)),
            pl.BlockSpec((8, 1280), lambda j: (0, 0)),
            pl.BlockSpec((3, NPAD), lambda j: (0, 0)),
        ],
        out_specs=[
            pl.BlockSpec((QB, 3), lambda j: (jnp.minimum(j, nblk - 1), 0)),
            pl.BlockSpec((QB, K), lambda j: (jnp.maximum(j - 1, 0), 0)),
        ],
        out_shape=[
            jax.ShapeDtypeStruct((MPAD, 3), jnp.float32),
            jax.ShapeDtypeStruct((MPAD, K), jnp.int32),
        ],
        scratch_shapes=[
            pltpu.VMEM((8, 1280), jnp.float32),
            pltpu.VMEM((QB, 3), jnp.float32),
            pltpu.VMEM((QB, NPAD), jnp.float32),
        ],
    )(px, py, pz, pt)


QB = 32


def _mlp_body(x_ref, w_ref, b_ref, out_ref):
    acc = jnp.dot(x_ref[:, :], w_ref[:, :], preferred_element_type=jnp.float32)
    out_ref[:, :] = jnp.maximum(acc + b_ref[:, :], 0.0)


def _run_mlp(x, w, b2):
    return pl.pallas_call(
        _mlp_body,
        grid=(10,),
        in_specs=[
            pl.BlockSpec((1000, IN_C), lambda i: (i, 0)),
            pl.BlockSpec((IN_C, OUT_C), lambda i: (0, 0)),
            pl.BlockSpec((1, OUT_C), lambda i: (0, 0)),
        ],
        out_specs=pl.BlockSpec((1000, OUT_C), lambda i: (i, 0)),
        out_shape=jax.ShapeDtypeStruct((N, OUT_C), jnp.float32),
    )(x, w, b2)


_Q_PER_TILE = MSC // 32
_Q_CHUNK = 8
_N_CHUNKS = _Q_PER_TILE // _Q_CHUNK


def _sc_gather_max(h, idx_flat):
    info = plsc.get_sparse_core_info()
    nc = info.num_cores

    mesh = plsc.VectorSubcoreMesh(core_axis_name="c", subcore_axis_name="s")

    @functools.partial(
        pl.kernel,
        mesh=mesh,
        out_type=jax.ShapeDtypeStruct((MSC, OUT_C), jnp.float32),
        scratch_types=[
            pltpu.VMEM((_Q_CHUNK * K,), jnp.int32),
            pltpu.VMEM((_Q_CHUNK * K, OUT_C), jnp.float32),
            pltpu.VMEM((_Q_CHUNK, OUT_C), jnp.float32),
            pltpu.SemaphoreType.DMA,
        ],
    )
    def k(h_hbm, idx_hbm, out_hbm, idx_v, rows_v, acc_v, sem):
        wid = lax.axis_index("s") * nc + lax.axis_index("c")
        base_q = wid * _Q_PER_TILE

        def chunk(c, _):
            qoff = base_q + c * _Q_CHUNK
            pltpu.sync_copy(idx_hbm.at[pl.ds(qoff * K, _Q_CHUNK * K)], idx_v)
            pltpu.async_copy(h_hbm.at[idx_v], rows_v, sem).wait()

            def per_query(q, __):
                row0 = q * K
                for d in range(OUT_C // 16):
                    ds = pl.ds(d * 16, 16)
                    acc = rows_v[row0, ds]
                    for r in range(1, K):
                        acc = jnp.maximum(acc, rows_v[row0 + r, ds])
                    acc_v[q, ds] = acc
                return 0

            lax.fori_loop(0, _Q_CHUNK, per_query, 0)
            pltpu.sync_copy(acc_v, out_hbm.at[pl.ds(qoff, _Q_CHUNK)])
            return 0

        lax.fori_loop(0, _N_CHUNKS, chunk, 0)

    return k(h, idx_flat)


def kernel(x, pos, batch, W, b):
    posx = jnp.pad(pos[:, 0], (0, NPAD - N), constant_values=1e3)
    posy = jnp.pad(pos[:, 1], (0, NPAD - N), constant_values=1e3)
    posz = jnp.pad(pos[:, 2], (0, NPAD - N), constant_values=1e3)
    px = posx.reshape(8, 1280)
    py = posy.reshape(8, 1280)
    pz = posz.reshape(8, 1280)
    pt = jnp.stack([posx, posy, posz])

    h = _run_mlp(x, W, b.reshape(1, OUT_C))

    subpos_pad, idx_pad = _run_fps_knn(px, py, pz, pt)
    sub_pos = subpos_pad[:M]
    idx = idx_pad[:M]

    idx_sc = jnp.zeros((MSC, K), jnp.int32).at[:M].set(idx).reshape(-1)
    x_out = _sc_gather_max(h, idx_sc)[:M]

    sub_batch = jnp.zeros((M,), jnp.int32)
    return (x_out, sub_pos, sub_batch)


# speedup vs baseline: 2.9845x; 1.0248x over previous
import functools

import jax
import jax.numpy as jnp
from jax import lax
from jax.experimental import pallas as pl
from jax.experimental.pallas import tpu as pltpu
from jax.experimental.pallas import tpu_sc as plsc

N = 10000
IN_C = 128
OUT_C = 256
K = 16
M = 2500

NPAD = 10240
MPAD = 2560
MSC = 2560

_BIGF = float(1e30)
_NEGF = float(-1e30)
_BIGI = 2**30


def _fps_step(px_ref, py_ref, pz_ref, dists_ref, flat, iotaq, t, carry):
    sx_a, sy_a, sz_a = carry
    dists = dists_ref[:, :]
    m = jnp.max(dists)
    nxt = jnp.min(jnp.where(dists == m, flat, _BIGI))
    selmask = flat == nxt
    px = px_ref[:, :]
    py = py_ref[:, :]
    pz = pz_ref[:, :]
    sx = jnp.sum(jnp.where(selmask, px, 0.0))
    sy = jnp.sum(jnp.where(selmask, py, 0.0))
    sz = jnp.sum(jnp.where(selmask, pz, 0.0))
    dx = px - sx
    dy = py - sy
    dz = pz - sz
    d = dx * dx + dy * dy + dz * dz
    dists_ref[:, :] = jnp.minimum(dists, d)
    sx_a = jnp.where(iotaq == t, sx, sx_a)
    sy_a = jnp.where(iotaq == t, sy, sy_a)
    sz_a = jnp.where(iotaq == t, sz, sz_a)
    return (sx_a, sy_a, sz_a)


def _fused_body(px_ref, py_ref, pz_ref, pt_ref, out_ref, idx_ref,
                dists_ref, qsave_ref, neg_ref):
    j = pl.program_id(0)
    nblk = MPAD // QB
    rows = jax.lax.broadcasted_iota(jnp.int32, (8, 1280), 0)
    cols = jax.lax.broadcasted_iota(jnp.int32, (8, 1280), 1)
    flat = rows * 1280 + cols
    valid = flat < N

    @pl.when(j == 0)
    def _init():
        dists_ref[:, :] = jnp.where(valid, _BIGF, _NEGF)

    qprev = qsave_ref[:, :]

    qx = qprev[:, 0:1]
    qy = qprev[:, 1:2]
    qz = qprev[:, 2:3]
    ptx = pt_ref[0:1, :]
    pty = pt_ref[1:2, :]
    ptz = pt_ref[2:3, :]
    ynorm = qx * qx + qy * qy + qz * qz
    xnorm = ptx * ptx + pty * pty + ptz * ptz
    g = jnp.dot(qprev, pt_ref[:, :], preferred_element_type=jnp.float32)
    neg_ref[:, :] = -((ynorm - 2.0 * g) + xnorm)

    iotaq = jax.lax.broadcasted_iota(jnp.int32, (QB, 1), 0)
    colmap = jax.lax.broadcasted_iota(jnp.int32, (QB, NPAD), 1)
    lanek = jax.lax.broadcasted_iota(jnp.int32, (QB, K), 1)

    def step(t, carry):
        sx_a, sy_a, sz_a, idxacc = carry
        fps_c = (sx_a, sy_a, sz_a)
        fps_c = _fps_step(px_ref, py_ref, pz_ref, dists_ref, flat, iotaq,
                          2 * t, fps_c)
        fps_c = _fps_step(px_ref, py_ref, pz_ref, dists_ref, flat, iotaq,
                          2 * t + 1, fps_c)
        sx_a, sy_a, sz_a = fps_c
        neg = neg_ref[:, :]
        m = jnp.max(neg, axis=1, keepdims=True)
        c = jnp.min(jnp.where(neg == m, colmap, _BIGI), axis=1, keepdims=True)
        neg_ref[:, :] = jnp.where(colmap == c, _NEGF, neg)
        idxacc = jnp.where(lanek == t, c, idxacc)
        return (sx_a, sy_a, sz_a, idxacc)

    z = jnp.zeros((QB, 1), jnp.float32)
    zi = jnp.zeros((QB, K), jnp.int32)
    sx_a, sy_a, sz_a, idxacc = lax.fori_loop(0, K, step, (z, z, z, zi))

    @pl.when(j < nblk)
    def _wq():
        qnew = jnp.concatenate([sx_a, sy_a, sz_a], axis=1)
        out_ref[:, :] = qnew
        qsave_ref[:, :] = qnew

    @pl.when(j >= 1)
    def _wi():
        idx_ref[:, :] = idxacc


def _run_fps_knn(px, py, pz, pt):
    nblk = MPAD // QB
    return pl.pallas_call(
        _fused_body,
        grid=(nblk + 1,),
        in_specs=[
            pl.BlockSpec((8, 1280), lambda j: (0, 0)),
            pl.BlockSpec((8, 1280), lambda j: (0, 0)),
            pl.BlockSpec((8, 1280), lambda j: (0, 0)),
            pl.BlockSpec((3, NPAD), lambda j: (0, 0)),
        ],
        out_specs=[
            pl.BlockSpec((QB, 3), lambda j: (jnp.minimum(j, nblk - 1), 0)),
            pl.BlockSpec((QB, K), lambda j: (jnp.maximum(j - 1, 0), 0)),
        ],
        out_shape=[
            jax.ShapeDtypeStruct((MPAD, 3), jnp.float32),
            jax.ShapeDtypeStruct((MPAD, K), jnp.int32),
        ],
        scratch_shapes=[
            pltpu.VMEM((8, 1280), jnp.float32),
            pltpu.VMEM((QB, 3), jnp.float32),
            pltpu.VMEM((QB, NPAD), jnp.float32),
        ],
    )(px, py, pz, pt)


QB = 32


def _mlp_body(x_ref, w_ref, b_ref, out_ref):
    acc = jnp.dot(x_ref[:, :], w_ref[:, :], preferred_element_type=jnp.float32)
    out_ref[:, :] = jnp.maximum(acc + b_ref[:, :], 0.0)


def _run_mlp(x, w, b2):
    return pl.pallas_call(
        _mlp_body,
        grid=(10,),
        in_specs=[
            pl.BlockSpec((1000, IN_C), lambda i: (i, 0)),
            pl.BlockSpec((IN_C, OUT_C), lambda i: (0, 0)),
            pl.BlockSpec((1, OUT_C), lambda i: (0, 0)),
        ],
        out_specs=pl.BlockSpec((1000, OUT_C), lambda i: (i, 0)),
        out_shape=jax.ShapeDtypeStruct((N, OUT_C), jnp.float32),
    )(x, w, b2)


_Q_PER_TILE = MSC // 32
_Q_CHUNK = 8
_N_CHUNKS = _Q_PER_TILE // _Q_CHUNK


def _sc_gather_max(h, idx_flat):
    info = plsc.get_sparse_core_info()
    nc = info.num_cores

    mesh = plsc.VectorSubcoreMesh(core_axis_name="c", subcore_axis_name="s")

    @functools.partial(
        pl.kernel,
        mesh=mesh,
        out_type=jax.ShapeDtypeStruct((MSC, OUT_C), jnp.float32),
        scratch_types=[
            pltpu.VMEM((_Q_CHUNK * K,), jnp.int32),
            pltpu.VMEM((_Q_CHUNK * K, OUT_C), jnp.float32),
            pltpu.VMEM((_Q_CHUNK, OUT_C), jnp.float32),
            pltpu.SemaphoreType.DMA,
        ],
    )
    def k(h_hbm, idx_hbm, out_hbm, idx_v, rows_v, acc_v, sem):
        wid = lax.axis_index("s") * nc + lax.axis_index("c")
        base_q = wid * _Q_PER_TILE

        def chunk(c, _):
            qoff = base_q + c * _Q_CHUNK
            pltpu.sync_copy(idx_hbm.at[pl.ds(qoff * K, _Q_CHUNK * K)], idx_v)
            pltpu.async_copy(h_hbm.at[idx_v], rows_v, sem).wait()

            def per_query(q, __):
                row0 = q * K
                for d in range(OUT_C // 16):
                    ds = pl.ds(d * 16, 16)
                    acc = rows_v[row0, ds]
                    for r in range(1, K):
                        acc = jnp.maximum(acc, rows_v[row0 + r, ds])
                    acc_v[q, ds] = acc
                return 0

            lax.fori_loop(0, _Q_CHUNK, per_query, 0)
            pltpu.sync_copy(acc_v, out_hbm.at[pl.ds(qoff, _Q_CHUNK)])
            return 0

        lax.fori_loop(0, _N_CHUNKS, chunk, 0)

    return k(h, idx_flat)


def kernel(x, pos, batch, W, b):
    posx = jnp.pad(pos[:, 0], (0, NPAD - N), constant_values=1e3)
    posy = jnp.pad(pos[:, 1], (0, NPAD - N), constant_values=1e3)
    posz = jnp.pad(pos[:, 2], (0, NPAD - N), constant_values=1e3)
    px = posx.reshape(8, 1280)
    py = posy.reshape(8, 1280)
    pz = posz.reshape(8, 1280)
    pt = jnp.stack([posx, posy, posz])

    h = _run_mlp(x, W, b.reshape(1, OUT_C))

    subpos_pad, idx_pad = _run_fps_knn(px, py, pz, pt)
    sub_pos = subpos_pad[:M]

    x_out = _sc_gather_max(h, idx_pad.reshape(-1))[:M]

    sub_batch = jnp.zeros((M,), jnp.int32)
    return (x_out, sub_pos, sub_batch)


# speedup vs baseline: 3.0519x; 1.0226x over previous
import functools

import jax
import jax.numpy as jnp
from jax import lax
from jax.experimental import pallas as pl
from jax.experimental.pallas import tpu as pltpu
from jax.experimental.pallas import tpu_sc as plsc

N = 10000
IN_C = 128
OUT_C = 256
K = 16
M = 2500

NPAD = 10240
MPAD = 2560
MSC = 2560

_BIGF = float(1e30)
_NEGF = float(-1e30)
_BIGI = 2**30


def _fps_step(px_ref, py_ref, pz_ref, dists_ref, flat, iotaq, t, carry):
    sx_a, sy_a, sz_a = carry
    dists = dists_ref[:, :]
    m = jnp.max(dists)
    nxt = jnp.min(jnp.where(dists == m, flat, _BIGI))
    selmask = flat == nxt
    px = px_ref[:, :]
    py = py_ref[:, :]
    pz = pz_ref[:, :]
    sx = jnp.sum(jnp.where(selmask, px, 0.0))
    sy = jnp.sum(jnp.where(selmask, py, 0.0))
    sz = jnp.sum(jnp.where(selmask, pz, 0.0))
    dx = px - sx
    dy = py - sy
    dz = pz - sz
    d = dx * dx + dy * dy + dz * dz
    dists_ref[:, :] = jnp.minimum(dists, d)
    sx_a = jnp.where(iotaq == t, sx, sx_a)
    sy_a = jnp.where(iotaq == t, sy, sy_a)
    sz_a = jnp.where(iotaq == t, sz, sz_a)
    return (sx_a, sy_a, sz_a)


def _fused_body(px_ref, py_ref, pz_ref, pt_ref, x_ref, w_ref, b_ref,
                out_ref, idx_ref, h_ref, dists_ref, qsave_ref, neg_ref):
    j = pl.program_id(0)
    nblk = MPAD // QB

    @pl.when(j < N // MLP_BLK)
    def _mlp():
        acc = jnp.dot(x_ref[:, :], w_ref[:, :],
                      preferred_element_type=jnp.float32)
        h_ref[:, :] = jnp.maximum(acc + b_ref[:, :], 0.0)
    rows = jax.lax.broadcasted_iota(jnp.int32, (8, 1280), 0)
    cols = jax.lax.broadcasted_iota(jnp.int32, (8, 1280), 1)
    flat = rows * 1280 + cols
    valid = flat < N

    @pl.when(j == 0)
    def _init():
        dists_ref[:, :] = jnp.where(valid, _BIGF, _NEGF)

    qprev = qsave_ref[:, :]

    qx = qprev[:, 0:1]
    qy = qprev[:, 1:2]
    qz = qprev[:, 2:3]
    ptx = pt_ref[0:1, :]
    pty = pt_ref[1:2, :]
    ptz = pt_ref[2:3, :]
    ynorm = qx * qx + qy * qy + qz * qz
    xnorm = ptx * ptx + pty * pty + ptz * ptz
    g = jnp.dot(qprev, pt_ref[:, :], preferred_element_type=jnp.float32)
    neg_ref[:, :] = -((ynorm - 2.0 * g) + xnorm)

    iotaq = jax.lax.broadcasted_iota(jnp.int32, (QB, 1), 0)
    colmap = jax.lax.broadcasted_iota(jnp.int32, (QB, NPAD), 1)
    lanek = jax.lax.broadcasted_iota(jnp.int32, (QB, K), 1)

    def step(t, carry):
        sx_a, sy_a, sz_a, idxacc = carry
        fps_c = (sx_a, sy_a, sz_a)
        fps_c = _fps_step(px_ref, py_ref, pz_ref, dists_ref, flat, iotaq,
                          2 * t, fps_c)
        fps_c = _fps_step(px_ref, py_ref, pz_ref, dists_ref, flat, iotaq,
                          2 * t + 1, fps_c)
        sx_a, sy_a, sz_a = fps_c
        neg = neg_ref[:, :]
        m = jnp.max(neg, axis=1, keepdims=True)
        c = jnp.min(jnp.where(neg == m, colmap, _BIGI), axis=1, keepdims=True)
        neg_ref[:, :] = jnp.where(colmap == c, _NEGF, neg)
        idxacc = jnp.where(lanek == t, c, idxacc)
        return (sx_a, sy_a, sz_a, idxacc)

    z = jnp.zeros((QB, 1), jnp.float32)
    zi = jnp.zeros((QB, K), jnp.int32)
    sx_a, sy_a, sz_a, idxacc = lax.fori_loop(0, K, step, (z, z, z, zi))

    @pl.when(j < nblk)
    def _wq():
        qnew = jnp.concatenate([sx_a, sy_a, sz_a], axis=1)
        out_ref[:, :] = qnew
        qsave_ref[:, :] = qnew

    @pl.when(j >= 1)
    def _wi():
        idx_ref[:, :] = idxacc


MLP_BLK = 1000


def _run_fps_knn(px, py, pz, pt, x, w, b2):
    nblk = MPAD // QB
    nmlp = N // MLP_BLK
    return pl.pallas_call(
        _fused_body,
        grid=(nblk + 1,),
        in_specs=[
            pl.BlockSpec((8, 1280), lambda j: (0, 0)),
            pl.BlockSpec((8, 1280), lambda j: (0, 0)),
            pl.BlockSpec((8, 1280), lambda j: (0, 0)),
            pl.BlockSpec((3, NPAD), lambda j: (0, 0)),
            pl.BlockSpec((MLP_BLK, IN_C), lambda j: (jnp.minimum(j, nmlp - 1), 0)),
            pl.BlockSpec((IN_C, OUT_C), lambda j: (0, 0)),
            pl.BlockSpec((1, OUT_C), lambda j: (0, 0)),
        ],
        out_specs=[
            pl.BlockSpec((QB, 3), lambda j: (jnp.minimum(j, nblk - 1), 0)),
            pl.BlockSpec((QB, K), lambda j: (jnp.maximum(j - 1, 0), 0)),
            pl.BlockSpec((MLP_BLK, OUT_C), lambda j: (jnp.minimum(j, nmlp - 1), 0)),
        ],
        out_shape=[
            jax.ShapeDtypeStruct((MPAD, 3), jnp.float32),
            jax.ShapeDtypeStruct((MPAD, K), jnp.int32),
            jax.ShapeDtypeStruct((N, OUT_C), jnp.float32),
        ],
        scratch_shapes=[
            pltpu.VMEM((8, 1280), jnp.float32),
            pltpu.VMEM((QB, 3), jnp.float32),
            pltpu.VMEM((QB, NPAD), jnp.float32),
        ],
    )(px, py, pz, pt, x, w, b2)


QB = 32


_Q_PER_TILE = MSC // 32
_Q_CHUNK = 8
_N_CHUNKS = _Q_PER_TILE // _Q_CHUNK


def _sc_gather_max(h, idx_flat):
    info = plsc.get_sparse_core_info()
    nc = info.num_cores

    mesh = plsc.VectorSubcoreMesh(core_axis_name="c", subcore_axis_name="s")

    @functools.partial(
        pl.kernel,
        mesh=mesh,
        out_type=jax.ShapeDtypeStruct((MSC, OUT_C), jnp.float32),
        scratch_types=[
            pltpu.VMEM((_Q_PER_TILE * K,), jnp.int32),
            pltpu.VMEM((_Q_CHUNK * K, OUT_C), jnp.float32),
            pltpu.VMEM((_Q_CHUNK * K, OUT_C), jnp.float32),
            pltpu.VMEM((_Q_PER_TILE, OUT_C), jnp.float32),
            pltpu.SemaphoreType.DMA,
            pltpu.SemaphoreType.DMA,
            pltpu.SemaphoreType.DMA,
        ],
    )
    def k(h_hbm, idx_hbm, out_hbm, idx_v, rows0_v, rows1_v, acc_v,
          sem0, sem1, semo):
        wid = lax.axis_index("s") * nc + lax.axis_index("c")
        base_q = wid * _Q_PER_TILE

        pltpu.sync_copy(idx_hbm.at[pl.ds(base_q * K, _Q_PER_TILE * K)], idx_v)

        bufs = [(rows0_v, sem0), (rows1_v, sem1)]
        copies = {}
        for c in range(min(2, _N_CHUNKS)):
            rows_v, sem = bufs[c % 2]
            copies[c] = pltpu.async_copy(
                h_hbm.at[idx_v.at[pl.ds(c * _Q_CHUNK * K, _Q_CHUNK * K)]],
                rows_v, sem)

        for c in range(_N_CHUNKS):
            rows_v, sem = bufs[c % 2]
            copies[c].wait()

            def per_query(q, __, rows_v=rows_v, c=c):
                row0 = q * K
                for d in range(OUT_C // 16):
                    ds = pl.ds(d * 16, 16)
                    acc = rows_v[row0, ds]
                    for r in range(1, K):
                        acc = jnp.maximum(acc, rows_v[row0 + r, ds])
                    acc_v[c * _Q_CHUNK + q, ds] = acc
                return 0

            lax.fori_loop(0, _Q_CHUNK, per_query, 0)

            nxt = c + 2
            if nxt < _N_CHUNKS:
                rows_n, sem_n = bufs[nxt % 2]
                copies[nxt] = pltpu.async_copy(
                    h_hbm.at[idx_v.at[pl.ds(nxt * _Q_CHUNK * K, _Q_CHUNK * K)]],
                    rows_n, sem_n)

        pltpu.async_copy(acc_v, out_hbm.at[pl.ds(base_q, _Q_PER_TILE)],
                         semo).wait()

    return k(h, idx_flat)


def kernel(x, pos, batch, W, b):
    posx = jnp.pad(pos[:, 0], (0, NPAD - N), constant_values=1e3)
    posy = jnp.pad(pos[:, 1], (0, NPAD - N), constant_values=1e3)
    posz = jnp.pad(pos[:, 2], (0, NPAD - N), constant_values=1e3)
    px = posx.reshape(8, 1280)
    py = posy.reshape(8, 1280)
    pz = posz.reshape(8, 1280)
    pt = jnp.stack([posx, posy, posz])

    subpos_pad, idx_pad, h = _run_fps_knn(px, py, pz, pt, x, W,
                                          b.reshape(1, OUT_C))
    sub_pos = subpos_pad[:M]

    x_out = _sc_gather_max(h, idx_pad.reshape(-1))[:M]

    sub_batch = jnp.zeros((M,), jnp.int32)
    return (x_out, sub_pos, sub_batch)


# speedup vs baseline: 3.1552x; 1.0338x over previous
import functools

import jax
import jax.numpy as jnp
from jax import lax
from jax.experimental import pallas as pl
from jax.experimental.pallas import tpu as pltpu
from jax.experimental.pallas import tpu_sc as plsc

N = 10000
IN_C = 128
OUT_C = 256
K = 16
M = 2500

NPAD = 10240
MPAD = 2560
MSC = 2560

_BIGF = float(1e30)
_NEGF = float(-1e30)
_BIGI = 2**30


def _fps_step(px_ref, py_ref, pz_ref, ps_ref, dists_ref, flat, iotaq, t, carry):
    sx_a, sy_a, sz_a = carry
    dists = dists_ref[:, :]
    m = jnp.max(dists)
    nxt = jnp.min(jnp.where(dists == m, flat, _BIGI))
    px = px_ref[:, :]
    py = py_ref[:, :]
    pz = pz_ref[:, :]
    sx = ps_ref[nxt, 0]
    sy = ps_ref[nxt, 1]
    sz = ps_ref[nxt, 2]
    dx = px - sx
    dy = py - sy
    dz = pz - sz
    d = dx * dx + dy * dy + dz * dz
    dists_ref[:, :] = jnp.minimum(dists, d)
    sx_a = jnp.where(iotaq == t, sx, sx_a)
    sy_a = jnp.where(iotaq == t, sy, sy_a)
    sz_a = jnp.where(iotaq == t, sz, sz_a)
    return (sx_a, sy_a, sz_a)


def _fused_body(px_ref, py_ref, pz_ref, pt_ref, ps_ref, x_ref, w_ref, b_ref,
                out_ref, idx_ref, h_ref, dists_ref, qsave_ref, neg_ref):
    j = pl.program_id(0)
    nblk = MPAD // QB

    @pl.when(j < N // MLP_BLK)
    def _mlp():
        acc = jnp.dot(x_ref[:, :], w_ref[:, :],
                      preferred_element_type=jnp.float32)
        h_ref[:, :] = jnp.maximum(acc + b_ref[:, :], 0.0)
    rows = jax.lax.broadcasted_iota(jnp.int32, (8, 1280), 0)
    cols = jax.lax.broadcasted_iota(jnp.int32, (8, 1280), 1)
    flat = rows * 1280 + cols
    valid = flat < N

    @pl.when(j == 0)
    def _init():
        dists_ref[:, :] = jnp.where(valid, _BIGF, _NEGF)

    qprev = qsave_ref[:, :]

    qx = qprev[:, 0:1]
    qy = qprev[:, 1:2]
    qz = qprev[:, 2:3]
    ptx = pt_ref[0:1, :]
    pty = pt_ref[1:2, :]
    ptz = pt_ref[2:3, :]
    ynorm = qx * qx + qy * qy + qz * qz
    xnorm = ptx * ptx + pty * pty + ptz * ptz
    g = jnp.dot(qprev, pt_ref[:, :], preferred_element_type=jnp.float32)
    neg_ref[:, :] = -((ynorm - 2.0 * g) + xnorm)

    iotaq = jax.lax.broadcasted_iota(jnp.int32, (QB, 1), 0)
    colmap = jax.lax.broadcasted_iota(jnp.int32, (QB, NPAD), 1)
    lanek = jax.lax.broadcasted_iota(jnp.int32, (QB, K), 1)

    def step(t, carry):
        sx_a, sy_a, sz_a, idxacc = carry
        fps_c = (sx_a, sy_a, sz_a)
        fps_c = _fps_step(px_ref, py_ref, pz_ref, ps_ref, dists_ref, flat,
                          iotaq, 2 * t, fps_c)
        fps_c = _fps_step(px_ref, py_ref, pz_ref, ps_ref, dists_ref, flat,
                          iotaq, 2 * t + 1, fps_c)
        sx_a, sy_a, sz_a = fps_c
        neg = neg_ref[:, :]
        m = jnp.max(neg, axis=1, keepdims=True)
        c = jnp.min(jnp.where(neg == m, colmap, _BIGI), axis=1, keepdims=True)
        neg_ref[:, :] = jnp.where(colmap == c, _NEGF, neg)
        idxacc = jnp.where(lanek == t, c, idxacc)
        return (sx_a, sy_a, sz_a, idxacc)

    z = jnp.zeros((QB, 1), jnp.float32)
    zi = jnp.zeros((QB, K), jnp.int32)
    sx_a, sy_a, sz_a, idxacc = lax.fori_loop(0, K, step, (z, z, z, zi))

    @pl.when(j < nblk)
    def _wq():
        qnew = jnp.concatenate([sx_a, sy_a, sz_a], axis=1)
        out_ref[:, :] = qnew
        qsave_ref[:, :] = qnew

    @pl.when(j >= 1)
    def _wi():
        idx_ref[:, :] = idxacc


MLP_BLK = 1000


def _run_fps_knn(px, py, pz, pt, ps, x, w, b2):
    nblk = MPAD // QB
    nmlp = N // MLP_BLK
    return pl.pallas_call(
        _fused_body,
        grid=(nblk + 1,),
        in_specs=[
            pl.BlockSpec((8, 1280), lambda j: (0, 0)),
            pl.BlockSpec((8, 1280), lambda j: (0, 0)),
            pl.BlockSpec((8, 1280), lambda j: (0, 0)),
            pl.BlockSpec((3, NPAD), lambda j: (0, 0)),
            pl.BlockSpec((NPAD, 3), lambda j: (0, 0)),
            pl.BlockSpec((MLP_BLK, IN_C), lambda j: (jnp.minimum(j, nmlp - 1), 0)),
            pl.BlockSpec((IN_C, OUT_C), lambda j: (0, 0)),
            pl.BlockSpec((1, OUT_C), lambda j: (0, 0)),
        ],
        out_specs=[
            pl.BlockSpec((QB, 3), lambda j: (jnp.minimum(j, nblk - 1), 0)),
            pl.BlockSpec((QB, K), lambda j: (jnp.maximum(j - 1, 0), 0)),
            pl.BlockSpec((MLP_BLK, OUT_C), lambda j: (jnp.minimum(j, nmlp - 1), 0)),
        ],
        out_shape=[
            jax.ShapeDtypeStruct((MPAD, 3), jnp.float32),
            jax.ShapeDtypeStruct((MPAD, K), jnp.int32),
            jax.ShapeDtypeStruct((N, OUT_C), jnp.float32),
        ],
        scratch_shapes=[
            pltpu.VMEM((8, 1280), jnp.float32),
            pltpu.VMEM((QB, 3), jnp.float32),
            pltpu.VMEM((QB, NPAD), jnp.float32),
        ],
    )(px, py, pz, pt, ps, x, w, b2)


QB = 32


_Q_PER_TILE = MSC // 32
_Q_CHUNK = 8
_N_CHUNKS = _Q_PER_TILE // _Q_CHUNK


def _sc_gather_max(h, idx_flat):
    info = plsc.get_sparse_core_info()
    nc = info.num_cores

    mesh = plsc.VectorSubcoreMesh(core_axis_name="c", subcore_axis_name="s")

    @functools.partial(
        pl.kernel,
        mesh=mesh,
        out_type=jax.ShapeDtypeStruct((MSC, OUT_C), jnp.float32),
        scratch_types=[
            pltpu.VMEM((_Q_PER_TILE * K,), jnp.int32),
            pltpu.VMEM((_Q_CHUNK * K, OUT_C), jnp.float32),
            pltpu.VMEM((_Q_CHUNK * K, OUT_C), jnp.float32),
            pltpu.VMEM((_Q_PER_TILE, OUT_C), jnp.float32),
            pltpu.SemaphoreType.DMA,
            pltpu.SemaphoreType.DMA,
            pltpu.SemaphoreType.DMA,
        ],
    )
    def k(h_hbm, idx_hbm, out_hbm, idx_v, rows0_v, rows1_v, acc_v,
          sem0, sem1, semo):
        wid = lax.axis_index("s") * nc + lax.axis_index("c")
        base_q = wid * _Q_PER_TILE

        pltpu.sync_copy(idx_hbm.at[pl.ds(base_q * K, _Q_PER_TILE * K)], idx_v)

        bufs = [(rows0_v, sem0), (rows1_v, sem1)]
        copies = {}
        for c in range(min(2, _N_CHUNKS)):
            rows_v, sem = bufs[c % 2]
            copies[c] = pltpu.async_copy(
                h_hbm.at[idx_v.at[pl.ds(c * _Q_CHUNK * K, _Q_CHUNK * K)]],
                rows_v, sem)

        for c in range(_N_CHUNKS):
            rows_v, sem = bufs[c % 2]
            copies[c].wait()

            def per_query(q, __, rows_v=rows_v, c=c):
                row0 = q * K
                for d in range(OUT_C // 16):
                    ds = pl.ds(d * 16, 16)
                    acc = rows_v[row0, ds]
                    for r in range(1, K):
                        acc = jnp.maximum(acc, rows_v[row0 + r, ds])
                    acc_v[c * _Q_CHUNK + q, ds] = acc
                return 0

            lax.fori_loop(0, _Q_CHUNK, per_query, 0)

            nxt = c + 2
            if nxt < _N_CHUNKS:
                rows_n, sem_n = bufs[nxt % 2]
                copies[nxt] = pltpu.async_copy(
                    h_hbm.at[idx_v.at[pl.ds(nxt * _Q_CHUNK * K, _Q_CHUNK * K)]],
                    rows_n, sem_n)

        pltpu.async_copy(acc_v, out_hbm.at[pl.ds(base_q, _Q_PER_TILE)],
                         semo).wait()

    return k(h, idx_flat)


def kernel(x, pos, batch, W, b):
    posx = jnp.pad(pos[:, 0], (0, NPAD - N), constant_values=1e3)
    posy = jnp.pad(pos[:, 1], (0, NPAD - N), constant_values=1e3)
    posz = jnp.pad(pos[:, 2], (0, NPAD - N), constant_values=1e3)
    px = posx.reshape(8, 1280)
    py = posy.reshape(8, 1280)
    pz = posz.reshape(8, 1280)
    pt = jnp.stack([posx, posy, posz])

    subpos_pad, idx_pad, h = _run_fps_knn(px, py, pz, pt, pt.T, x, W,
                                          b.reshape(1, OUT_C))
    sub_pos = subpos_pad[:M]

    x_out = _sc_gather_max(h, idx_pad.reshape(-1))[:M]

    sub_batch = jnp.zeros((M,), jnp.int32)
    return (x_out, sub_pos, sub_batch)


# speedup vs baseline: 3.1745x; 1.0061x over previous
import functools

import jax
import jax.numpy as jnp
from jax import lax
from jax.experimental import pallas as pl
from jax.experimental.pallas import tpu as pltpu
from jax.experimental.pallas import tpu_sc as plsc

N = 10000
IN_C = 128
OUT_C = 256
K = 16
M = 2500

NPAD = 10240
MPAD = 2560
MSC = 2560

_BIGF = float(1e30)
_NEGF = float(-1e30)
_BIGI = 2**30


def _vreg_argmax(d):
    nm = d[:, 0:128]
    nv = jnp.zeros((8, 128), jnp.int32)
    for v in range(1, 10):
        sv = d[:, v * 128:(v + 1) * 128]
        upd = sv > nm
        nv = jnp.where(upd, v, nv)
        nm = jnp.where(upd, sv, nm)
    return nm, nv


def _fps_step(px_ref, py_ref, pz_ref, ps_ref, dists_ref, m1_ref, v1_ref,
              iotaq, t, carry):
    sx_a, sy_a, sz_a = carry
    sub8 = jax.lax.broadcasted_iota(jnp.int32, (8, 128), 0)
    lane128 = jax.lax.broadcasted_iota(jnp.int32, (8, 128), 1)
    m1 = m1_ref[:, :]
    v1 = v1_ref[:, :]
    m = jnp.max(m1)
    flatv = sub8 * 1280 + v1 * 128 + lane128
    nxt = jnp.min(jnp.where(m1 == m, flatv, _BIGI))
    px = px_ref[:, :]
    py = py_ref[:, :]
    pz = pz_ref[:, :]
    sx = ps_ref[nxt, 0]
    sy = ps_ref[nxt, 1]
    sz = ps_ref[nxt, 2]
    dx = px - sx
    dy = py - sy
    dz = pz - sz
    d = dx * dx + dy * dy + dz * dz
    newd = jnp.minimum(dists_ref[:, :], d)
    dists_ref[:, :] = newd
    nm, nv = _vreg_argmax(newd)
    m1_ref[:, :] = nm
    v1_ref[:, :] = nv
    sx_a = jnp.where(iotaq == t, sx, sx_a)
    sy_a = jnp.where(iotaq == t, sy, sy_a)
    sz_a = jnp.where(iotaq == t, sz, sz_a)
    return (sx_a, sy_a, sz_a)


def _fused_body(px_ref, py_ref, pz_ref, pt_ref, ps_ref, x_ref, w_ref, b_ref,
                out_ref, idx_ref, h_ref, dists_ref, qsave_ref, neg_ref,
                m1_ref, v1_ref):
    j = pl.program_id(0)
    nblk = MPAD // QB

    @pl.when(j < N // MLP_BLK)
    def _mlp():
        acc = jnp.dot(x_ref[:, :], w_ref[:, :],
                      preferred_element_type=jnp.float32)
        h_ref[:, :] = jnp.maximum(acc + b_ref[:, :], 0.0)
    rows = jax.lax.broadcasted_iota(jnp.int32, (8, 1280), 0)
    cols = jax.lax.broadcasted_iota(jnp.int32, (8, 1280), 1)
    flat = rows * 1280 + cols
    valid = flat < N

    @pl.when(j == 0)
    def _init():
        d0 = jnp.where(valid, _BIGF, _NEGF)
        dists_ref[:, :] = d0
        nm, nv = _vreg_argmax(d0)
        m1_ref[:, :] = nm
        v1_ref[:, :] = nv

    qprev = qsave_ref[:, :]

    qx = qprev[:, 0:1]
    qy = qprev[:, 1:2]
    qz = qprev[:, 2:3]
    ptx = pt_ref[0:1, :]
    pty = pt_ref[1:2, :]
    ptz = pt_ref[2:3, :]
    ynorm = qx * qx + qy * qy + qz * qz
    xnorm = ptx * ptx + pty * pty + ptz * ptz
    g = jnp.dot(qprev, pt_ref[:, :], preferred_element_type=jnp.float32)
    neg_ref[:, :] = -((ynorm - 2.0 * g) + xnorm)

    iotaq = jax.lax.broadcasted_iota(jnp.int32, (QB, 1), 0)
    colmap = jax.lax.broadcasted_iota(jnp.int32, (QB, NPAD), 1)
    lanek = jax.lax.broadcasted_iota(jnp.int32, (QB, K), 1)

    def step(t, carry):
        sx_a, sy_a, sz_a, idxacc = carry
        fps_c = (sx_a, sy_a, sz_a)
        fps_c = _fps_step(px_ref, py_ref, pz_ref, ps_ref, dists_ref,
                          m1_ref, v1_ref, iotaq, 2 * t, fps_c)
        fps_c = _fps_step(px_ref, py_ref, pz_ref, ps_ref, dists_ref,
                          m1_ref, v1_ref, iotaq, 2 * t + 1, fps_c)
        sx_a, sy_a, sz_a = fps_c
        neg = neg_ref[:, :]
        m = jnp.max(neg, axis=1, keepdims=True)
        c = jnp.min(jnp.where(neg == m, colmap, _BIGI), axis=1, keepdims=True)
        neg_ref[:, :] = jnp.where(colmap == c, _NEGF, neg)
        idxacc = jnp.where(lanek == t, c, idxacc)
        return (sx_a, sy_a, sz_a, idxacc)

    z = jnp.zeros((QB, 1), jnp.float32)
    zi = jnp.zeros((QB, K), jnp.int32)
    sx_a, sy_a, sz_a, idxacc = lax.fori_loop(0, K, step, (z, z, z, zi))

    @pl.when(j < nblk)
    def _wq():
        qnew = jnp.concatenate([sx_a, sy_a, sz_a], axis=1)
        out_ref[:, :] = qnew
        qsave_ref[:, :] = qnew

    @pl.when(j >= 1)
    def _wi():
        idx_ref[:, :] = idxacc


MLP_BLK = 1000


def _run_fps_knn(px, py, pz, pt, ps, x, w, b2):
    nblk = MPAD // QB
    nmlp = N // MLP_BLK
    return pl.pallas_call(
        _fused_body,
        grid=(nblk + 1,),
        in_specs=[
            pl.BlockSpec((8, 1280), lambda j: (0, 0)),
            pl.BlockSpec((8, 1280), lambda j: (0, 0)),
            pl.BlockSpec((8, 1280), lambda j: (0, 0)),
            pl.BlockSpec((3, NPAD), lambda j: (0, 0)),
            pl.BlockSpec((NPAD, 3), lambda j: (0, 0)),
            pl.BlockSpec((MLP_BLK, IN_C), lambda j: (jnp.minimum(j, nmlp - 1), 0)),
            pl.BlockSpec((IN_C, OUT_C), lambda j: (0, 0)),
            pl.BlockSpec((1, OUT_C), lambda j: (0, 0)),
        ],
        out_specs=[
            pl.BlockSpec((QB, 3), lambda j: (jnp.minimum(j, nblk - 1), 0)),
            pl.BlockSpec((QB, K), lambda j: (jnp.maximum(j - 1, 0), 0)),
            pl.BlockSpec((MLP_BLK, OUT_C), lambda j: (jnp.minimum(j, nmlp - 1), 0)),
        ],
        out_shape=[
            jax.ShapeDtypeStruct((MPAD, 3), jnp.float32),
            jax.ShapeDtypeStruct((MPAD, K), jnp.int32),
            jax.ShapeDtypeStruct((N, OUT_C), jnp.float32),
        ],
        scratch_shapes=[
            pltpu.VMEM((8, 1280), jnp.float32),
            pltpu.VMEM((QB, 3), jnp.float32),
            pltpu.VMEM((QB, NPAD), jnp.float32),
            pltpu.VMEM((8, 128), jnp.float32),
            pltpu.VMEM((8, 128), jnp.int32),
        ],
    )(px, py, pz, pt, ps, x, w, b2)


QB = 32


_Q_PER_TILE = MSC // 32
_Q_CHUNK = 8
_N_CHUNKS = _Q_PER_TILE // _Q_CHUNK


def _sc_gather_max(h, idx_flat):
    info = plsc.get_sparse_core_info()
    nc = info.num_cores

    mesh = plsc.VectorSubcoreMesh(core_axis_name="c", subcore_axis_name="s")

    @functools.partial(
        pl.kernel,
        mesh=mesh,
        out_type=jax.ShapeDtypeStruct((MSC, OUT_C), jnp.float32),
        scratch_types=[
            pltpu.VMEM((_Q_PER_TILE * K,), jnp.int32),
            pltpu.VMEM((_Q_CHUNK * K, OUT_C), jnp.float32),
            pltpu.VMEM((_Q_CHUNK * K, OUT_C), jnp.float32),
            pltpu.VMEM((_Q_PER_TILE, OUT_C), jnp.float32),
            pltpu.SemaphoreType.DMA,
            pltpu.SemaphoreType.DMA,
            pltpu.SemaphoreType.DMA,
        ],
    )
    def k(h_hbm, idx_hbm, out_hbm, idx_v, rows0_v, rows1_v, acc_v,
          sem0, sem1, semo):
        wid = lax.axis_index("s") * nc + lax.axis_index("c")
        base_q = wid * _Q_PER_TILE

        pltpu.sync_copy(idx_hbm.at[pl.ds(base_q * K, _Q_PER_TILE * K)], idx_v)

        bufs = [(rows0_v, sem0), (rows1_v, sem1)]
        copies = {}
        for c in range(min(2, _N_CHUNKS)):
            rows_v, sem = bufs[c % 2]
            copies[c] = pltpu.async_copy(
                h_hbm.at[idx_v.at[pl.ds(c * _Q_CHUNK * K, _Q_CHUNK * K)]],
                rows_v, sem)

        for c in range(_N_CHUNKS):
            rows_v, sem = bufs[c % 2]
            copies[c].wait()

            def per_query(q, __, rows_v=rows_v, c=c):
                row0 = q * K
                for d in range(OUT_C // 16):
                    ds = pl.ds(d * 16, 16)
                    acc = rows_v[row0, ds]
                    for r in range(1, K):
                        acc = jnp.maximum(acc, rows_v[row0 + r, ds])
                    acc_v[c * _Q_CHUNK + q, ds] = acc
                return 0

            lax.fori_loop(0, _Q_CHUNK, per_query, 0)

            nxt = c + 2
            if nxt < _N_CHUNKS:
                rows_n, sem_n = bufs[nxt % 2]
                copies[nxt] = pltpu.async_copy(
                    h_hbm.at[idx_v.at[pl.ds(nxt * _Q_CHUNK * K, _Q_CHUNK * K)]],
                    rows_n, sem_n)

        pltpu.async_copy(acc_v, out_hbm.at[pl.ds(base_q, _Q_PER_TILE)],
                         semo).wait()

    return k(h, idx_flat)


def kernel(x, pos, batch, W, b):
    posx = jnp.pad(pos[:, 0], (0, NPAD - N), constant_values=1e3)
    posy = jnp.pad(pos[:, 1], (0, NPAD - N), constant_values=1e3)
    posz = jnp.pad(pos[:, 2], (0, NPAD - N), constant_values=1e3)
    px = posx.reshape(8, 1280)
    py = posy.reshape(8, 1280)
    pz = posz.reshape(8, 1280)
    pt = jnp.stack([posx, posy, posz])

    subpos_pad, idx_pad, h = _run_fps_knn(px, py, pz, pt, pt.T, x, W,
                                          b.reshape(1, OUT_C))
    sub_pos = subpos_pad[:M]

    x_out = _sc_gather_max(h, idx_pad.reshape(-1))[:M]

    sub_batch = jnp.zeros((M,), jnp.int32)
    return (x_out, sub_pos, sub_batch)
